# Initial kernel scaffold; baseline (speedup 1.0000x reference)
import jax
import jax.numpy as jnp
from jax.experimental import pallas as pl


def kernel(x, edge_index, gamma, beta, W1_0, b1_0, W2_0, b2_0, W1_1, b1_1, W2_1, b2_1, W1_2, b1_2, W2_2, b2_2):
    raise NotImplementedError("write your pallas kernel here")


# speedup vs baseline: 7.6953x; 7.6953x over previous
import functools

import jax
import jax.numpy as jnp
from jax import lax
from jax.experimental import pallas as pl
from jax.experimental.pallas import tpu as pltpu
from jax.experimental.pallas import tpu_sc as plsc

_NC = 2
_NS = 16
_NW = _NC * _NS
_CH = 128


def _make_agg(N, E, D):
    n_chunks = E // _CH
    cpw = n_chunks // _NW
    extra = n_chunks - cpw * _NW
    CR = 80
    n_row_chunks = N // CR
    passes = (n_row_chunks + _NS - 1) // _NS
    mesh = plsc.VectorSubcoreMesh(core_axis_name="c", subcore_axis_name="s")

    @functools.partial(
        pl.kernel,
        out_type=jax.ShapeDtypeStruct((_NC, N, D), jnp.float32),
        mesh=mesh,
        scratch_types=[
            pltpu.VMEM((cpw, _CH), jnp.int32),
            pltpu.VMEM((cpw, _CH), jnp.int32),
            pltpu.VMEM((_CH,), jnp.int32),
            pltpu.VMEM((_CH,), jnp.int32),
            pltpu.VMEM((_CH, D), jnp.float32),
            pltpu.VMEM((CR, D), jnp.float32),
            pltpu.VMEM_SHARED((N, D), jnp.float32),
            pltpu.SemaphoreType.DMA,
        ],
    )
    def agg(x_hbm, src_hbm, dst_hbm, srcx_hbm, dstx_hbm, zeros_hbm, out_hbm,
            src_v, dst_v, srcx_v, dstx_v, rows_v, stage_v, acc_sh, sem):
        cid = lax.axis_index("c")
        sid = lax.axis_index("s")
        wid = sid * _NC + cid

        pltpu.sync_copy(zeros_hbm, stage_v)
        for t in range(passes):
            ch = t * _NS + sid
            @pl.when(ch < n_row_chunks)
            def _():
                pltpu.sync_copy(stage_v, acc_sh.at[pl.ds(ch * CR, CR)])

        pltpu.sync_copy(src_hbm.at[wid], src_v)
        pltpu.sync_copy(dst_hbm.at[wid], dst_v)
        plsc.subcore_barrier()

        @pl.loop(0, cpw)
        def _edges(g):
            pltpu.async_copy(x_hbm.at[src_v.at[g]], rows_v, sem).wait()
            pltpu.sync_copy(rows_v, acc_sh.at[dst_v.at[g]], add=True)

        @pl.when(wid < extra)
        def _tail():
            pltpu.sync_copy(srcx_hbm.at[pl.ds(wid * _CH, _CH)], srcx_v)
            pltpu.sync_copy(dstx_hbm.at[pl.ds(wid * _CH, _CH)], dstx_v)
            pltpu.async_copy(x_hbm.at[srcx_v], rows_v, sem).wait()
            pltpu.sync_copy(rows_v, acc_sh.at[dstx_v], add=True)

        plsc.subcore_barrier()

        for t in range(passes):
            ch = t * _NS + sid
            @pl.when(ch < n_row_chunks)
            def _():
                pltpu.sync_copy(acc_sh.at[pl.ds(ch * CR, CR)], stage_v)
                pltpu.sync_copy(stage_v, out_hbm.at[cid, pl.ds(ch * CR, CR)])

    return agg


def _mlp_body(x_ref, p0_ref, p1_ref, w1_ref, b1_ref, w2_ref, b2_ref,
              h_ref, sum_ref, sq_ref):
    i = pl.program_id(0)
    s = x_ref[...] + p0_ref[...] + p1_ref[...]
    h = jnp.maximum(
        jnp.dot(s, w1_ref[...], preferred_element_type=jnp.float32) + b1_ref[...], 0.0)
    h = jnp.maximum(
        jnp.dot(h, w2_ref[...], preferred_element_type=jnp.float32) + b2_ref[...], 0.0)
    h_ref[...] = h

    @pl.when(i == 0)
    def _():
        sum_ref[...] = jnp.zeros_like(sum_ref)
        sq_ref[...] = jnp.zeros_like(sq_ref)

    sum_ref[0:1, :] = sum_ref[0:1, :] + jnp.sum(h, axis=0, keepdims=True)
    sq_ref[0:1, :] = sq_ref[0:1, :] + jnp.sum(h * h, axis=0, keepdims=True)


def _bn_body(n_rows, h_ref, sum_ref, sq_ref, g_ref, b_ref, o_ref):
    mean = sum_ref[0:1, :] / n_rows
    var = sq_ref[0:1, :] / n_rows - mean * mean
    rstd = lax.rsqrt(var + 1e-5)
    o_ref[...] = (h_ref[...] - mean) * rstd * g_ref[...] + b_ref[...]


def kernel(x, edge_index, gamma, beta,
           W1_0, b1_0, W2_0, b2_0,
           W1_1, b1_1, W2_1, b2_1,
           W1_2, b1_2, W2_2, b2_2):
    N, D = x.shape
    E = edge_index.shape[1]
    H = W1_0.shape[1]
    n_chunks = E // _CH
    cpw = n_chunks // _NW
    n_main = _NW * cpw

    src_main = edge_index[0, :n_main * _CH].reshape(_NW, cpw, _CH)
    dst_main = edge_index[1, :n_main * _CH].reshape(_NW, cpw, _CH)
    src_x = edge_index[0, n_main * _CH:]
    dst_x = edge_index[1, n_main * _CH:]
    zeros_h = jnp.zeros((80, D), jnp.float32)

    agg = _make_agg(N, E, D)

    NB = 10
    B = N // NB
    row_spec = pl.BlockSpec((B, D), lambda i: (i, 0))
    full = lambda shape: pl.BlockSpec(shape, lambda i: (0, 0))

    mlp_call = pl.pallas_call(
        _mlp_body,
        grid=(NB,),
        in_specs=[row_spec, row_spec, row_spec,
                  full((D, H)), full((1, H)), full((H, H)), full((1, H))],
        out_specs=[pl.BlockSpec((B, H), lambda i: (i, 0)),
                   full((8, H)), full((8, H))],
        out_shape=[jax.ShapeDtypeStruct((N, H), jnp.float32),
                   jax.ShapeDtypeStruct((8, H), jnp.float32),
                   jax.ShapeDtypeStruct((8, H), jnp.float32)],
    )
    bn_call = pl.pallas_call(
        functools.partial(_bn_body, float(N)),
        grid=(NB,),
        in_specs=[pl.BlockSpec((B, H), lambda i: (i, 0)),
                  full((8, H)), full((8, H)), full((1, H)), full((1, H))],
        out_specs=pl.BlockSpec((B, H), lambda i: (i, 0)),
        out_shape=jax.ShapeDtypeStruct((N, H), jnp.float32),
    )

    g_r = gamma.reshape(1, H)
    be_r = beta.reshape(1, H)
    params = [(W1_0, b1_0, W2_0, b2_0),
              (W1_1, b1_1, W2_1, b2_1),
              (W1_2, b1_2, W2_2, b2_2)]

    cur = x
    for (W1, b1, W2, b2) in params:
        parts = agg(cur, src_main, dst_main, src_x, dst_x, zeros_h)
        h, ssum, ssq = mlp_call(cur, parts[0], parts[1],
                                W1, b1.reshape(1, H), W2, b2.reshape(1, H))
        cur = bn_call(h, ssum, ssq, g_r, be_r)
    return cur


# speedup vs baseline: 9.8379x; 1.2784x over previous
import functools

import jax
import jax.numpy as jnp
from jax import lax
from jax.experimental import pallas as pl
from jax.experimental.pallas import tpu as pltpu
from jax.experimental.pallas import tpu_sc as plsc

_NC = 2
_NS = 16
_NW = _NC * _NS
_CH = 128


def _make_agg(N, E, D):
    n_chunks = E // _CH
    cpw = n_chunks // _NW
    extra = n_chunks - cpw * _NW
    CR = 80
    n_row_chunks = N // CR
    passes = (n_row_chunks + _NS - 1) // _NS
    mesh = plsc.VectorSubcoreMesh(core_axis_name="c", subcore_axis_name="s")

    @functools.partial(
        pl.kernel,
        out_type=jax.ShapeDtypeStruct((_NC, N, D), jnp.float32),
        mesh=mesh,
        scratch_types=[
            pltpu.VMEM((_CH,), jnp.int32),
            pltpu.VMEM((_CH,), jnp.int32),
            pltpu.VMEM((_CH,), jnp.int32),
            pltpu.VMEM((_CH,), jnp.int32),
            pltpu.VMEM((_CH, D), jnp.float32),
            pltpu.VMEM((_CH, D), jnp.float32),
            pltpu.VMEM((CR, D), jnp.float32),
            pltpu.VMEM_SHARED((N, D), jnp.float32),
            pltpu.SemaphoreType.DMA,
            pltpu.SemaphoreType.DMA,
            pltpu.SemaphoreType.DMA,
            pltpu.SemaphoreType.DMA,
        ],
    )
    def agg(x_hbm, src_hbm, dst_hbm, zeros_hbm, out_hbm,
            srci0, srci1, dsti0, dsti1, rows0, rows1, stage_v, acc_sh,
            gsem0, gsem1, isem0, isem1):
        cid = lax.axis_index("c")
        sid = lax.axis_index("s")
        wid = sid * _NC + cid

        pltpu.sync_copy(zeros_hbm, stage_v)
        for t in range(passes):
            ch = t * _NS + sid
            @pl.when(ch < n_row_chunks)
            def _():
                pltpu.sync_copy(stage_v, acc_sh.at[pl.ds(ch * CR, CR)])

        plsc.subcore_barrier()

        bufs = ((srci0, dsti0, rows0, gsem0, isem0),
                (srci1, dsti1, rows1, gsem1, isem1))
        e0 = wid * cpw

        def idx_copy(chunk, sbuf, dbuf, sem):
            off = (e0 + chunk) * _CH
            return (pltpu.make_async_copy(src_hbm.at[pl.ds(off, _CH)], sbuf, sem),
                    pltpu.make_async_copy(dst_hbm.at[pl.ds(off, _CH)], dbuf, sem))

        for c in idx_copy(0, srci0, dsti0, isem0):
            c.start()
            c.wait()
        pltpu.async_copy(x_hbm.at[srci0], rows0, gsem0)
        for c in idx_copy(1, srci1, dsti1, isem1):
            c.start()

        @pl.loop(0, cpw // 2)
        def _edges(go):
            for b in range(2):
                g = 2 * go + b
                sbuf, dbuf, rbuf, gsem, isem = bufs[b]
                nsbuf, ndbuf, nrbuf, ngsem, nisem = bufs[1 - b]

                pltpu.make_async_copy(x_hbm.at[sbuf], rbuf, gsem).wait()

                @pl.when(g + 1 < cpw)
                def _():
                    for c in idx_copy(g + 1, nsbuf, ndbuf, nisem):
                        c.wait()
                    pltpu.async_copy(x_hbm.at[nsbuf], nrbuf, ngsem)

                pltpu.sync_copy(rbuf, acc_sh.at[dbuf], add=True)

                @pl.when(g + 2 < cpw)
                def _():
                    for c in idx_copy(g + 2, sbuf, dbuf, isem):
                        c.start()

        @pl.when(wid < extra)
        def _tail():
            off = (_NW * cpw + wid) * _CH
            pltpu.sync_copy(src_hbm.at[pl.ds(off, _CH)], srci0)
            pltpu.sync_copy(dst_hbm.at[pl.ds(off, _CH)], dsti0)
            pltpu.async_copy(x_hbm.at[srci0], rows0, gsem0).wait()
            pltpu.sync_copy(rows0, acc_sh.at[dsti0], add=True)

        plsc.subcore_barrier()

        for t in range(passes):
            ch = t * _NS + sid
            @pl.when(ch < n_row_chunks)
            def _():
                pltpu.sync_copy(acc_sh.at[pl.ds(ch * CR, CR)], stage_v)
                pltpu.sync_copy(stage_v, out_hbm.at[cid, pl.ds(ch * CR, CR)])

    return agg


def _mlp_body(x_ref, p0_ref, p1_ref, w1_ref, b1_ref, w2_ref, b2_ref,
              h_ref, sum_ref, sq_ref):
    i = pl.program_id(0)
    s = x_ref[...] + p0_ref[...] + p1_ref[...]
    h = jnp.maximum(
        jnp.dot(s, w1_ref[...], preferred_element_type=jnp.float32) + b1_ref[...], 0.0)
    h = jnp.maximum(
        jnp.dot(h, w2_ref[...], preferred_element_type=jnp.float32) + b2_ref[...], 0.0)
    h_ref[...] = h

    @pl.when(i == 0)
    def _():
        sum_ref[...] = jnp.zeros_like(sum_ref)
        sq_ref[...] = jnp.zeros_like(sq_ref)

    sum_ref[0:1, :] = sum_ref[0:1, :] + jnp.sum(h, axis=0, keepdims=True)
    sq_ref[0:1, :] = sq_ref[0:1, :] + jnp.sum(h * h, axis=0, keepdims=True)


def _bn_body(n_rows, h_ref, sum_ref, sq_ref, g_ref, b_ref, o_ref):
    mean = sum_ref[0:1, :] / n_rows
    var = sq_ref[0:1, :] / n_rows - mean * mean
    rstd = lax.rsqrt(var + 1e-5)
    o_ref[...] = (h_ref[...] - mean) * rstd * g_ref[...] + b_ref[...]


def kernel(x, edge_index, gamma, beta,
           W1_0, b1_0, W2_0, b2_0,
           W1_1, b1_1, W2_1, b2_1,
           W1_2, b1_2, W2_2, b2_2):
    N, D = x.shape
    E = edge_index.shape[1]
    H = W1_0.shape[1]
    n_chunks = E // _CH
    cpw = n_chunks // _NW
    n_main = _NW * cpw

    src_1d = edge_index[0]
    dst_1d = edge_index[1]
    zeros_h = jnp.zeros((80, D), jnp.float32)

    agg = _make_agg(N, E, D)

    NB = 10
    B = N // NB
    row_spec = pl.BlockSpec((B, D), lambda i: (i, 0))
    full = lambda shape: pl.BlockSpec(shape, lambda i: (0, 0))

    mlp_call = pl.pallas_call(
        _mlp_body,
        grid=(NB,),
        in_specs=[row_spec, row_spec, row_spec,
                  full((D, H)), full((1, H)), full((H, H)), full((1, H))],
        out_specs=[pl.BlockSpec((B, H), lambda i: (i, 0)),
                   full((8, H)), full((8, H))],
        out_shape=[jax.ShapeDtypeStruct((N, H), jnp.float32),
                   jax.ShapeDtypeStruct((8, H), jnp.float32),
                   jax.ShapeDtypeStruct((8, H), jnp.float32)],
    )
    bn_call = pl.pallas_call(
        functools.partial(_bn_body, float(N)),
        grid=(NB,),
        in_specs=[pl.BlockSpec((B, H), lambda i: (i, 0)),
                  full((8, H)), full((8, H)), full((1, H)), full((1, H))],
        out_specs=pl.BlockSpec((B, H), lambda i: (i, 0)),
        out_shape=jax.ShapeDtypeStruct((N, H), jnp.float32),
    )

    g_r = gamma.reshape(1, H)
    be_r = beta.reshape(1, H)
    params = [(W1_0, b1_0, W2_0, b2_0),
              (W1_1, b1_1, W2_1, b2_1),
              (W1_2, b1_2, W2_2, b2_2)]

    cur = x
    for (W1, b1, W2, b2) in params:
        parts = agg(cur, src_1d, dst_1d, zeros_h)
        h, ssum, ssq = mlp_call(cur, parts[0], parts[1],
                                W1, b1.reshape(1, H), W2, b2.reshape(1, H))
        cur = bn_call(h, ssum, ssq, g_r, be_r)
    return cur
